# Initial kernel scaffold; baseline (speedup 1.0000x reference)
import functools

import numpy as np
import jax
import jax.numpy as jnp
from jax import lax
from jax.experimental import pallas as pl
from jax.experimental.pallas import tpu as pltpu
from jax.experimental.pallas import tpu_sc as plsc

H = 4
HD = 8
DM = 32
NMOD = 4


def _build_masks():
    m1 = np.zeros((512, 64), np.float32)
    d1 = np.zeros((64, 16), np.float32)
    d2 = np.zeros((16, 64), np.float32)
    xp = np.zeros((64, 512), np.float32)
    s1 = np.zeros((512, 128), np.float32)
    for i in range(4):
        for j in range(4):
            p = i * 4 + j
            for h in range(H):
                e = p * 4 + h
                f = i * 4 + h
                d1[e, f] = 1.0
                d2[f, e] = 1.0
                for d in range(HD):
                    m1[p * 32 + h * 8 + d, e] = 1.0
                    xp[e, i * 128 + j * 32 + h * 8 + d] = 1.0
                    s1[i * 128 + j * 32 + h * 8 + d, i * 32 + h * 8 + d] = 1.0
    return m1, d1, d2, xp, s1


_M1, _D1, _D2, _XP, _S1 = _build_masks()


def _fuse_body(x_ref, dp_ref, wcat_ref, bcat_ref, ipw_ref, ipb_ref,
               opw_ref, opb_ref, w1_ref, m1_ref, d1_ref, d2_ref, xp_ref,
               s1_ref, z1_ref, y1_ref, dis_ref):
    xb = x_ref[...]
    p = jnp.maximum(xb @ wcat_ref[...] + bcat_ref[...], 0.0)
    ipw = ipw_ref[...]
    ipb = ipb_ref[...]
    qs, ks, vs = [], [], []
    for m in range(NMOD):
        qkv = p[:, m * 32:(m + 1) * 32] @ ipw + ipb
        qs.append(qkv[:, 0:32])
        ks.append(qkv[:, 32:64])
        vs.append(qkv[:, 64:96])
    qq = jnp.concatenate([qs[i] for i in range(4) for _ in range(4)], axis=1)
    kk = jnp.concatenate(ks * 4, axis=1)
    prod = qq * kk
    s64 = (prod @ m1_ref[...]) * (1.0 / np.sqrt(HD))
    rowmax = jnp.max(s64, axis=1, keepdims=True)
    e64 = jnp.exp(s64 - rowmax)
    den = e64 @ d1_ref[...]
    den64 = den @ d2_ref[...]
    attn = e64 / den64
    aa = attn @ xp_ref[...]
    vv = jnp.concatenate(vs * 4, axis=1)
    o = (aa * vv) @ s1_ref[...]
    osum = o[:, 0:32] + o[:, 32:64] + o[:, 64:96] + o[:, 96:128]
    fused = osum @ (opw_ref[...] * 0.25) + opb_ref[...]
    z1 = fused @ w1_ref[...]
    dp = dp_ref[...]
    deg = 1.0 + dp[0] + dp[1]
    dis = lax.rsqrt(deg)
    z1_ref[...] = z1
    dis_ref[...] = dis
    y1_ref[0, :, :] = z1[:, 0:16] * dis
    y1_ref[1, :, :] = z1[:, 16:32] * dis


def _fuse_call(x, dp, wcat, bcat, ipw, ipb, opw, opb, w1, interpret=False):
    n = x.shape[0]
    t = 1000 if n % 1000 == 0 else n
    g = n // t
    consts = [jnp.asarray(c) for c in (_M1, _D1, _D2, _XP, _S1)]
    full = lambda a: pl.BlockSpec(a.shape, lambda i: (0,) * a.ndim)
    return pl.pallas_call(
        _fuse_body,
        grid=(g,),
        in_specs=[
            pl.BlockSpec((t, x.shape[1]), lambda i: (i, 0)),
            pl.BlockSpec((2, t, 16), lambda i: (0, i, 0)),
            full(wcat), full(bcat), full(ipw), full(ipb), full(opw),
            full(opb), full(w1), full(consts[0]), full(consts[1]),
            full(consts[2]), full(consts[3]), full(consts[4]),
        ],
        out_specs=[
            pl.BlockSpec((t, DM), lambda i: (i, 0)),
            pl.BlockSpec((2, t, 16), lambda i: (0, i, 0)),
            pl.BlockSpec((t, 16), lambda i: (i, 0)),
        ],
        out_shape=[
            jax.ShapeDtypeStruct((n, DM), jnp.float32),
            jax.ShapeDtypeStruct((2, n, 16), jnp.float32),
            jax.ShapeDtypeStruct((n, 16), jnp.float32),
        ],
        interpret=interpret,
    )(x, dp, wcat, bcat, ipw, ipb, opw, opb, w1, *consts)


def _mid_body(agg_ref, z1_ref, dis_ref, b1_ref, w2_ref, z2_ref, y2_ref):
    agg = jnp.concatenate([agg_ref[0], agg_ref[1]], axis=1)
    z1 = z1_ref[...]
    dis = dis_ref[...]
    dis32 = jnp.concatenate([dis, dis], axis=1)
    h = jnp.maximum(agg * dis32 + z1 * (dis32 * dis32) + b1_ref[...], 0.0)
    z2 = h @ w2_ref[...]
    z2_ref[...] = z2
    y2_ref[0, :, :] = z2[:, 0:16] * dis
    y2_ref[1, :, :] = z2[:, 16:32] * dis


def _mid_call(agg, z1, dis, b1, w2, interpret=False):
    n = z1.shape[0]
    t = 2000 if n % 2000 == 0 else n
    g = n // t
    full = lambda a: pl.BlockSpec(a.shape, lambda i: (0,) * a.ndim)
    return pl.pallas_call(
        _mid_body,
        grid=(g,),
        in_specs=[
            pl.BlockSpec((2, t, 16), lambda i: (0, i, 0)),
            pl.BlockSpec((t, DM), lambda i: (i, 0)),
            pl.BlockSpec((t, 16), lambda i: (i, 0)),
            full(b1), full(w2),
        ],
        out_specs=[
            pl.BlockSpec((t, DM), lambda i: (i, 0)),
            pl.BlockSpec((2, t, 16), lambda i: (0, i, 0)),
        ],
        out_shape=[
            jax.ShapeDtypeStruct((n, DM), jnp.float32),
            jax.ShapeDtypeStruct((2, n, 16), jnp.float32),
        ],
        interpret=interpret,
    )(agg, z1, dis, b1, w2)


def _fin_body(agg_ref, z2_ref, dis_ref, b2_ref, out_ref):
    agg = jnp.concatenate([agg_ref[0], agg_ref[1]], axis=1)
    dis = dis_ref[...]
    dis32 = jnp.concatenate([dis, dis], axis=1)
    out_ref[...] = agg * dis32 + z2_ref[...] * (dis32 * dis32) + b2_ref[...]


def _fin_call(agg, z2, dis, b2, interpret=False):
    n = z2.shape[0]
    t = 2000 if n % 2000 == 0 else n
    g = n // t
    full = lambda a: pl.BlockSpec(a.shape, lambda i: (0,) * a.ndim)
    return pl.pallas_call(
        _fin_body,
        grid=(g,),
        in_specs=[
            pl.BlockSpec((2, t, 16), lambda i: (0, i, 0)),
            pl.BlockSpec((t, DM), lambda i: (i, 0)),
            pl.BlockSpec((t, 16), lambda i: (i, 0)),
            full(b2),
        ],
        out_specs=pl.BlockSpec((t, DM), lambda i: (i, 0)),
        out_shape=jax.ShapeDtypeStruct((n, DM), jnp.float32),
        interpret=interpret,
    )(agg, z2, dis, b2)


_CHROWS = 16
_LANES = 128


def _deg_body(nsub, rows_per_w, acc_rows, n, dst_ref, z_ref, ones_ref,
              out_ref, dbuf, ones_v, acc, sem):
    c = lax.axis_index("c")
    s = lax.axis_index("s")
    zr = acc_rows // nsub
    pltpu.sync_copy(z_ref, acc.at[pl.ds(s * zr, zr)])
    pltpu.sync_copy(ones_ref, ones_v)
    plsc.subcore_barrier()
    w = s * 2 + c
    base = w * rows_per_w
    nchunks = rows_per_w // _CHROWS

    def chunk(g, carry):
        r0 = base + g * _CHROWS
        pltpu.sync_copy(dst_ref.at[pl.ds(r0, _CHROWS)], dbuf)
        cps = [pltpu.async_copy(ones_v, acc.at[dbuf.at[j]], sem, add=True)
               for j in range(_CHROWS)]
        for cp in cps:
            cp.wait()
        return carry

    lax.fori_loop(0, nchunks, chunk, 0)
    plsc.subcore_barrier()
    orow = n // nsub
    pltpu.sync_copy(acc.at[pl.ds(s * orow, orow)],
                    out_ref.at[c, pl.ds(s * orow, orow)])


def _deg_call(dstp, zrows, ones, n, acc_rows):
    nsub = 16
    total_rows = dstp.shape[0]
    rows_per_w = total_rows // 32
    mesh = plsc.VectorSubcoreMesh(core_axis_name="c", subcore_axis_name="s")
    body = functools.partial(_deg_body, nsub, rows_per_w, acc_rows, n)
    return pl.kernel(
        body,
        out_type=jax.ShapeDtypeStruct((2, n, 16), jnp.float32),
        mesh=mesh,
        scratch_types=[
            pltpu.VMEM((_CHROWS, _LANES), jnp.int32),
            pltpu.VMEM((_LANES, 16), jnp.float32),
            pltpu.VMEM_SHARED((acc_rows, 16), jnp.float32),
            pltpu.SemaphoreType.DMA,
        ],
    )(dstp, zrows, ones)


def _gather_scatter_body(nsub, rows_per_sub, acc_rows, n, y_ref, srcadj_ref,
                         dst_ref, z_ref, out_ref, sbuf, dbuf, rows, acc, sem,
                         sem2):
    c = lax.axis_index("c")
    s = lax.axis_index("s")
    zr = acc_rows // nsub
    pltpu.sync_copy(z_ref, acc.at[pl.ds(s * zr, zr)])
    plsc.subcore_barrier()
    base = s * rows_per_sub
    nchunks = rows_per_sub // _CHROWS

    def chunk(g, carry):
        r0 = base + g * _CHROWS
        pltpu.sync_copy(srcadj_ref.at[c, pl.ds(r0, _CHROWS)], sbuf)
        pltpu.sync_copy(dst_ref.at[pl.ds(r0, _CHROWS)], dbuf)
        gs = [pltpu.async_copy(y_ref.at[sbuf.at[j]], rows.at[j], sem)
              for j in range(_CHROWS)]
        for cp in gs:
            cp.wait()
        ss = [pltpu.async_copy(rows.at[j], acc.at[dbuf.at[j]], sem2, add=True)
              for j in range(_CHROWS)]
        for cp in ss:
            cp.wait()
        return carry

    lax.fori_loop(0, nchunks, chunk, 0)
    plsc.subcore_barrier()
    orow = n // nsub
    pltpu.sync_copy(acc.at[pl.ds(s * orow, orow)],
                    out_ref.at[c, pl.ds(s * orow, orow)])


def _gather_scatter_call(yflat, srcadj, dstp, zrows, n, acc_rows):
    nsub = 16
    total_rows = dstp.shape[0]
    rows_per_sub = total_rows // nsub
    mesh = plsc.VectorSubcoreMesh(core_axis_name="c", subcore_axis_name="s")
    body = functools.partial(_gather_scatter_body, nsub, rows_per_sub,
                             acc_rows, n)
    return pl.kernel(
        body,
        out_type=jax.ShapeDtypeStruct((2, n, 16), jnp.float32),
        mesh=mesh,
        scratch_types=[
            pltpu.VMEM((_CHROWS, _LANES), jnp.int32),
            pltpu.VMEM((_CHROWS, _LANES), jnp.int32),
            pltpu.VMEM((_CHROWS, _LANES, 16), jnp.float32),
            pltpu.VMEM_SHARED((acc_rows, 16), jnp.float32),
            pltpu.SemaphoreType.DMA,
            pltpu.SemaphoreType.DMA,
        ],
    )(yflat, srcadj, dstp, zrows)


def kernel(x, edge_index, go_W, go_b, react_W, react_b, mesh_W, mesh_b,
           gwas_W, gwas_b, in_proj_W, in_proj_b, out_proj_W, out_proj_b,
           conv1_W, conv1_b, conv2_W, conv2_b):
    n = x.shape[0]
    ne = edge_index.shape[1]
    go, react, mesh_d = go_W.shape[0], react_W.shape[0], mesh_W.shape[0]
    fdim = x.shape[1]

    wcat = jnp.zeros((fdim, 4 * DM), jnp.float32)
    wcat = wcat.at[:go, 0:32].set(go_W)
    wcat = wcat.at[go:go + react, 32:64].set(react_W)
    wcat = wcat.at[go + react:go + react + mesh_d, 64:96].set(mesh_W)
    wcat = wcat.at[go + react + mesh_d:, 96:128].set(gwas_W)
    bcat = jnp.concatenate([go_b, react_b, mesh_b, gwas_b])[None, :]
    ipb = in_proj_b[None, :]
    opb = out_proj_b[None, :]
    b1 = conv1_b[None, :]
    b2 = conv2_b[None, :]

    nep = ((ne + 65535) // 65536) * 65536
    acc_rows = -(-(n + 1024) // (16 * 512)) * (16 * 512)
    nsink = acc_rows - n
    pe = nep - ne
    src = edge_index[0]
    dst = edge_index[1]
    padi = jnp.arange(pe, dtype=jnp.int32)
    srcp = jnp.concatenate([src, padi % n])
    dstp = jnp.concatenate([dst, n + (padi % nsink)])
    srcadj = jnp.stack([srcp, srcp + n]).reshape(2, nep // _LANES, _LANES)
    dstp = dstp.reshape(nep // _LANES, _LANES)
    zrows = jnp.zeros((acc_rows // 16, 16), jnp.float32)
    ones = jnp.ones((_LANES, 16), jnp.float32)

    dp = _deg_call(dstp, zrows, ones, n, acc_rows)
    z1, y1, dis = _fuse_call(x, dp, wcat, bcat, in_proj_W, ipb,
                             out_proj_W, opb, conv1_W)
    agg1 = _gather_scatter_call(y1.reshape(2 * n, 16), srcadj, dstp,
                                zrows, n, acc_rows)
    z2, y2 = _mid_call(agg1, z1, dis, b1, conv2_W)
    agg2 = _gather_scatter_call(y2.reshape(2 * n, 16), srcadj, dstp,
                                zrows, n, acc_rows)
    return _fin_call(agg2, z2, dis, b2)


# speedup vs baseline: 21.8178x; 21.8178x over previous
import functools

import numpy as np
import jax
import jax.numpy as jnp
from jax import lax
from jax.experimental import pallas as pl
from jax.experimental.pallas import tpu as pltpu
from jax.experimental.pallas import tpu_sc as plsc

H = 4
HD = 8
DM = 32
NMOD = 4


def _build_masks():
    m1 = np.zeros((512, 64), np.float32)
    d1 = np.zeros((64, 16), np.float32)
    d2 = np.zeros((16, 64), np.float32)
    xp = np.zeros((64, 512), np.float32)
    s1 = np.zeros((512, 128), np.float32)
    for i in range(4):
        for j in range(4):
            p = i * 4 + j
            for h in range(H):
                e = p * 4 + h
                f = i * 4 + h
                d1[e, f] = 1.0
                d2[f, e] = 1.0
                for d in range(HD):
                    m1[p * 32 + h * 8 + d, e] = 1.0
                    xp[e, i * 128 + j * 32 + h * 8 + d] = 1.0
                    s1[i * 128 + j * 32 + h * 8 + d, i * 32 + h * 8 + d] = 1.0
    return m1, d1, d2, xp, s1


_M1, _D1, _D2, _XP, _S1 = _build_masks()


def _fuse_body(x_ref, dp_ref, wcat_ref, bcat_ref, ipw_ref, ipb_ref,
               opw_ref, opb_ref, w1_ref, m1_ref, d1_ref, d2_ref, xp_ref,
               s1_ref, z1_ref, y1_ref, dis_ref):
    xb = x_ref[...]
    p = jnp.maximum(xb @ wcat_ref[...] + bcat_ref[...], 0.0)
    ipw = ipw_ref[...]
    ipb = ipb_ref[...]
    qs, ks, vs = [], [], []
    for m in range(NMOD):
        qkv = p[:, m * 32:(m + 1) * 32] @ ipw + ipb
        qs.append(qkv[:, 0:32])
        ks.append(qkv[:, 32:64])
        vs.append(qkv[:, 64:96])
    qq = jnp.concatenate([qs[i] for i in range(4) for _ in range(4)], axis=1)
    kk = jnp.concatenate(ks * 4, axis=1)
    prod = qq * kk
    s64 = (prod @ m1_ref[...]) * (1.0 / np.sqrt(HD))
    rowmax = jnp.max(s64, axis=1, keepdims=True)
    e64 = jnp.exp(s64 - rowmax)
    den = e64 @ d1_ref[...]
    den64 = den @ d2_ref[...]
    attn = e64 / den64
    aa = attn @ xp_ref[...]
    vv = jnp.concatenate(vs * 4, axis=1)
    o = (aa * vv) @ s1_ref[...]
    osum = o[:, 0:32] + o[:, 32:64] + o[:, 64:96] + o[:, 96:128]
    fused = osum @ (opw_ref[...] * 0.25) + opb_ref[...]
    z1 = fused @ w1_ref[...]
    dp = dp_ref[...]
    deg = 1.0 + dp[0] + dp[1]
    dis = lax.rsqrt(deg)
    z1_ref[...] = z1
    dis_ref[...] = dis
    y1_ref[0, :, :] = z1[:, 0:16] * dis
    y1_ref[1, :, :] = z1[:, 16:32] * dis


def _fuse_call(x, dp, wcat, bcat, ipw, ipb, opw, opb, w1, interpret=False):
    n = x.shape[0]
    t = 1000 if n % 1000 == 0 else n
    g = n // t
    consts = [jnp.asarray(c) for c in (_M1, _D1, _D2, _XP, _S1)]
    full = lambda a: pl.BlockSpec(a.shape, lambda i: (0,) * a.ndim)
    return pl.pallas_call(
        _fuse_body,
        grid=(g,),
        in_specs=[
            pl.BlockSpec((t, x.shape[1]), lambda i: (i, 0)),
            pl.BlockSpec((2, t, 16), lambda i: (0, i, 0)),
            full(wcat), full(bcat), full(ipw), full(ipb), full(opw),
            full(opb), full(w1), full(consts[0]), full(consts[1]),
            full(consts[2]), full(consts[3]), full(consts[4]),
        ],
        out_specs=[
            pl.BlockSpec((t, DM), lambda i: (i, 0)),
            pl.BlockSpec((2, t, 16), lambda i: (0, i, 0)),
            pl.BlockSpec((t, 16), lambda i: (i, 0)),
        ],
        out_shape=[
            jax.ShapeDtypeStruct((n, DM), jnp.float32),
            jax.ShapeDtypeStruct((2, n, 16), jnp.float32),
            jax.ShapeDtypeStruct((n, 16), jnp.float32),
        ],
        interpret=interpret,
    )(x, dp, wcat, bcat, ipw, ipb, opw, opb, w1, *consts)


def _mid_body(agg_ref, z1_ref, dis_ref, b1_ref, w2_ref, z2_ref, y2_ref):
    agg = jnp.concatenate([agg_ref[0], agg_ref[1]], axis=1)
    z1 = z1_ref[...]
    dis = dis_ref[...]
    dis32 = jnp.concatenate([dis, dis], axis=1)
    h = jnp.maximum(agg * dis32 + z1 * (dis32 * dis32) + b1_ref[...], 0.0)
    z2 = h @ w2_ref[...]
    z2_ref[...] = z2
    y2_ref[0, :, :] = z2[:, 0:16] * dis
    y2_ref[1, :, :] = z2[:, 16:32] * dis


def _mid_call(agg, z1, dis, b1, w2, interpret=False):
    n = z1.shape[0]
    t = 2000 if n % 2000 == 0 else n
    g = n // t
    full = lambda a: pl.BlockSpec(a.shape, lambda i: (0,) * a.ndim)
    return pl.pallas_call(
        _mid_body,
        grid=(g,),
        in_specs=[
            pl.BlockSpec((2, t, 16), lambda i: (0, i, 0)),
            pl.BlockSpec((t, DM), lambda i: (i, 0)),
            pl.BlockSpec((t, 16), lambda i: (i, 0)),
            full(b1), full(w2),
        ],
        out_specs=[
            pl.BlockSpec((t, DM), lambda i: (i, 0)),
            pl.BlockSpec((2, t, 16), lambda i: (0, i, 0)),
        ],
        out_shape=[
            jax.ShapeDtypeStruct((n, DM), jnp.float32),
            jax.ShapeDtypeStruct((2, n, 16), jnp.float32),
        ],
        interpret=interpret,
    )(agg, z1, dis, b1, w2)


def _fin_body(agg_ref, z2_ref, dis_ref, b2_ref, out_ref):
    agg = jnp.concatenate([agg_ref[0], agg_ref[1]], axis=1)
    dis = dis_ref[...]
    dis32 = jnp.concatenate([dis, dis], axis=1)
    out_ref[...] = agg * dis32 + z2_ref[...] * (dis32 * dis32) + b2_ref[...]


def _fin_call(agg, z2, dis, b2, interpret=False):
    n = z2.shape[0]
    t = 2000 if n % 2000 == 0 else n
    g = n // t
    full = lambda a: pl.BlockSpec(a.shape, lambda i: (0,) * a.ndim)
    return pl.pallas_call(
        _fin_body,
        grid=(g,),
        in_specs=[
            pl.BlockSpec((2, t, 16), lambda i: (0, i, 0)),
            pl.BlockSpec((t, DM), lambda i: (i, 0)),
            pl.BlockSpec((t, 16), lambda i: (i, 0)),
            full(b2),
        ],
        out_specs=pl.BlockSpec((t, DM), lambda i: (i, 0)),
        out_shape=jax.ShapeDtypeStruct((n, DM), jnp.float32),
        interpret=interpret,
    )(agg, z2, dis, b2)


_CHROWS = 8
_LANES = 128


def _deg_body(nsub, rows_per_w, acc_rows, n, dst_ref, z_ref, ones_ref,
              out_ref, dbuf, ones_v, acc, sem):
    c = lax.axis_index("c")
    s = lax.axis_index("s")
    zr = acc_rows // nsub
    pltpu.sync_copy(z_ref, acc.at[pl.ds(s * zr, zr)])
    pltpu.sync_copy(ones_ref, ones_v)
    plsc.subcore_barrier()
    w = s * 2 + c
    base = w * rows_per_w
    nchunks = rows_per_w // _CHROWS

    def chunk(g, carry):
        r0 = base + g * _CHROWS
        pltpu.sync_copy(dst_ref.at[pl.ds(r0, _CHROWS)], dbuf)
        cps = [pltpu.async_copy(ones_v, acc.at[dbuf.at[j]], sem, add=True)
               for j in range(_CHROWS)]
        for cp in cps:
            cp.wait()
        return carry

    lax.fori_loop(0, nchunks, chunk, 0)
    plsc.subcore_barrier()
    _copy_out(acc, out_ref, c, s, n, nsub)


def _copy_out(acc, out_ref, c, s, n, nsub):
    orow = -(-n // (nsub * 8)) * 8
    start = jnp.minimum(s * orow, n - orow)
    pltpu.sync_copy(acc.at[pl.ds(start, orow)],
                    out_ref.at[c, pl.ds(start, orow)])


def _deg_call(dstp, zrows, ones, n, acc_rows):
    nsub = 16
    total_rows = dstp.shape[0]
    rows_per_w = total_rows // 32
    mesh = plsc.VectorSubcoreMesh(core_axis_name="c", subcore_axis_name="s")
    body = functools.partial(_deg_body, nsub, rows_per_w, acc_rows, n)
    return pl.kernel(
        body,
        out_type=jax.ShapeDtypeStruct((2, n, 16), jnp.float32),
        mesh=mesh,
        scratch_types=[
            pltpu.VMEM((_CHROWS, _LANES), jnp.int32),
            pltpu.VMEM((_LANES, 16), jnp.float32),
            pltpu.VMEM_SHARED((acc_rows, 16), jnp.float32),
            pltpu.SemaphoreType.DMA,
        ],
        compiler_params=pltpu.CompilerParams(use_tc_tiling_on_sc=False),
    )(dstp, zrows, ones)


def _gather_scatter_body(nsub, rows_per_sub, acc_rows, n, y_ref, srcadj_ref,
                         dst_ref, z_ref, out_ref, sbuf, dbuf, rows, acc, sem,
                         sem2):
    c = lax.axis_index("c")
    s = lax.axis_index("s")
    zr = acc_rows // nsub
    pltpu.sync_copy(z_ref, acc.at[pl.ds(s * zr, zr)])
    plsc.subcore_barrier()
    base = s * rows_per_sub
    nchunks = rows_per_sub // _CHROWS

    def chunk(g, carry):
        r0 = base + g * _CHROWS
        pltpu.sync_copy(srcadj_ref.at[c, pl.ds(r0, _CHROWS)], sbuf)
        pltpu.sync_copy(dst_ref.at[pl.ds(r0, _CHROWS)], dbuf)
        gs = [pltpu.async_copy(y_ref.at[sbuf.at[j]], rows.at[j], sem)
              for j in range(_CHROWS)]
        for cp in gs:
            cp.wait()
        ss = [pltpu.async_copy(rows.at[j], acc.at[dbuf.at[j]], sem2, add=True)
              for j in range(_CHROWS)]
        for cp in ss:
            cp.wait()
        return carry

    lax.fori_loop(0, nchunks, chunk, 0)
    plsc.subcore_barrier()
    _copy_out(acc, out_ref, c, s, n, nsub)


def _gather_scatter_call(yflat, srcadj, dstp, zrows, n, acc_rows):
    nsub = 16
    total_rows = dstp.shape[0]
    rows_per_sub = total_rows // nsub
    mesh = plsc.VectorSubcoreMesh(core_axis_name="c", subcore_axis_name="s")
    body = functools.partial(_gather_scatter_body, nsub, rows_per_sub,
                             acc_rows, n)
    return pl.kernel(
        body,
        out_type=jax.ShapeDtypeStruct((2, n, 16), jnp.float32),
        mesh=mesh,
        scratch_types=[
            pltpu.VMEM((_CHROWS, _LANES), jnp.int32),
            pltpu.VMEM((_CHROWS, _LANES), jnp.int32),
            pltpu.VMEM((_CHROWS, _LANES, 16), jnp.float32),
            pltpu.VMEM_SHARED((acc_rows, 16), jnp.float32),
            pltpu.SemaphoreType.DMA,
            pltpu.SemaphoreType.DMA,
        ],
        compiler_params=pltpu.CompilerParams(use_tc_tiling_on_sc=False),
    )(yflat, srcadj, dstp, zrows)


def kernel(x, edge_index, go_W, go_b, react_W, react_b, mesh_W, mesh_b,
           gwas_W, gwas_b, in_proj_W, in_proj_b, out_proj_W, out_proj_b,
           conv1_W, conv1_b, conv2_W, conv2_b):
    n = x.shape[0]
    ne = edge_index.shape[1]
    go, react, mesh_d = go_W.shape[0], react_W.shape[0], mesh_W.shape[0]
    fdim = x.shape[1]

    wcat = jnp.zeros((fdim, 4 * DM), jnp.float32)
    wcat = wcat.at[:go, 0:32].set(go_W)
    wcat = wcat.at[go:go + react, 32:64].set(react_W)
    wcat = wcat.at[go + react:go + react + mesh_d, 64:96].set(mesh_W)
    wcat = wcat.at[go + react + mesh_d:, 96:128].set(gwas_W)
    bcat = jnp.concatenate([go_b, react_b, mesh_b, gwas_b])[None, :]
    ipb = in_proj_b[None, :]
    opb = out_proj_b[None, :]
    b1 = conv1_b[None, :]
    b2 = conv2_b[None, :]

    nep = ((ne + 65535) // 65536) * 65536
    acc_rows = -(-(n + 2048) // 2048) * 2048
    nsink = acc_rows - n
    pe = nep - ne
    src = edge_index[0]
    dst = edge_index[1]
    padi = jnp.arange(pe, dtype=jnp.int32)
    srcp = jnp.concatenate([src, padi % n])
    dstp = jnp.concatenate([dst, n + (padi % nsink)])
    srcadj = jnp.stack([srcp, srcp + n]).reshape(2, nep // _LANES, _LANES)
    dstp = dstp.reshape(nep // _LANES, _LANES)
    zrows = jnp.zeros((acc_rows // 16, 16), jnp.float32)
    ones = jnp.ones((_LANES, 16), jnp.float32)

    dp = _deg_call(dstp, zrows, ones, n, acc_rows)
    z1, y1, dis = _fuse_call(x, dp, wcat, bcat, in_proj_W, ipb,
                             out_proj_W, opb, conv1_W)
    agg1 = _gather_scatter_call(y1.reshape(2 * n, 16), srcadj, dstp,
                                zrows, n, acc_rows)
    z2, y2 = _mid_call(agg1, z1, dis, b1, conv2_W)
    agg2 = _gather_scatter_call(y2.reshape(2 * n, 16), srcadj, dstp,
                                zrows, n, acc_rows)
    return _fin_call(agg2, z2, dis, b2)
